# baseline (speedup 1.0000x reference)
import functools

import jax
import jax.numpy as jnp
from jax import lax
from jax.experimental import pallas as pl
from jax.experimental.pallas import tpu as pltpu
from jax.experimental.pallas import tpu_sc as plsc

_BLOCK = 4096
_NW = 32
_E = 64


def _scores_kernel(x_ref, w1_ref, b1_ref, w2_ref, b2_ref, s_ref):
    h = jnp.maximum(
        jnp.dot(x_ref[...], w1_ref[...], preferred_element_type=jnp.float32)
        + b1_ref[...],
        0.0,
    )
    s_ref[...] = (
        jnp.dot(h, w2_ref[...], preferred_element_type=jnp.float32) + b2_ref[...]
    )


def _scores(x, W1, b1, W2, b2):
    n = x.shape[0]
    return pl.pallas_call(
        _scores_kernel,
        grid=(n // _BLOCK,),
        in_specs=[
            pl.BlockSpec((_BLOCK, x.shape[1]), lambda i: (i, 0)),
            pl.BlockSpec(W1.shape, lambda i: (0, 0)),
            pl.BlockSpec(b1.shape, lambda i: (0,)),
            pl.BlockSpec(W2.shape, lambda i: (0, 0)),
            pl.BlockSpec(b2.shape, lambda i: (0,)),
        ],
        out_specs=pl.BlockSpec((_BLOCK, _E), lambda i: (i, 0)),
        out_shape=jax.ShapeDtypeStruct((n, _E), jnp.float32),
    )(x, W1, b1, W2, b2)


def _sc_top2(scores):
    n = scores.shape[0] // _E
    rpw = n // _NW
    groups = rpw // 16

    mesh = plsc.VectorSubcoreMesh(core_axis_name="c", subcore_axis_name="s")

    @functools.partial(
        pl.kernel,
        out_type=[
            jax.ShapeDtypeStruct((n * 2,), jnp.int32),
            jax.ShapeDtypeStruct((n * 2,), jnp.float32),
        ],
        mesh=mesh,
        scratch_types=[
            pltpu.VMEM((rpw * _E,), jnp.float32),
            pltpu.VMEM((rpw * 2,), jnp.int32),
            pltpu.VMEM((rpw * 2,), jnp.float32),
        ],
        compiler_params=pltpu.CompilerParams(needs_layout_passes=False),
    )
    def sc_kernel(s_hbm, idx_hbm, prob_hbm, s_v, idx_v, prob_v):
        wid = lax.axis_index("s") * 2 + lax.axis_index("c")
        base = wid * rpw
        pltpu.sync_copy(s_hbm.at[pl.ds(base * _E, rpw * _E)], s_v)

        def group(g, carry):
            row = g * 16 + lax.iota(jnp.int32, 16)
            rbase = row * _E
            m0 = jnp.full((16,), -jnp.inf, jnp.float32)
            m1 = jnp.full((16,), -jnp.inf, jnp.float32)
            i0 = jnp.zeros((16,), jnp.int32)
            i1 = jnp.zeros((16,), jnp.int32)
            for j in range(_E):
                col = jnp.full((16,), j, jnp.int32)
                v = plsc.load_gather(s_v, [rbase + j])
                gt0 = v > m0
                gt1 = v > m1
                m1 = jnp.where(gt0, m0, jnp.where(gt1, v, m1))
                i1 = jnp.where(gt0, i0, jnp.where(gt1, col, i1))
                m0 = jnp.where(gt0, v, m0)
                i0 = jnp.where(gt0, col, i0)
            d = jnp.zeros((16,), jnp.float32)
            for j in range(_E):
                v = plsc.load_gather(s_v, [rbase + j])
                d = d + jnp.exp(v - m0)
            p0 = 1.0 / d
            p1 = jnp.exp(m1 - m0) / d
            plsc.store_scatter(idx_v, [row * 2], i0)
            plsc.store_scatter(idx_v, [row * 2 + 1], i1)
            plsc.store_scatter(prob_v, [row * 2], p0)
            plsc.store_scatter(prob_v, [row * 2 + 1], p1)
            return carry

        lax.fori_loop(0, groups, group, 0)
        pltpu.sync_copy(idx_v, idx_hbm.at[pl.ds(base * 2, rpw * 2)])
        pltpu.sync_copy(prob_v, prob_hbm.at[pl.ds(base * 2, rpw * 2)])

    return sc_kernel(scores)


def kernel(x, W1, b1, W2, b2):
    s = _scores(x, W1, b1, W2, b2)
    idx_flat, prob_flat = _sc_top2(s.reshape(-1))
    n = x.shape[0]
    return idx_flat.reshape(n, 2), prob_flat.reshape(n, 2)


# speedup vs baseline: 2.4640x; 2.4640x over previous
import jax
import jax.numpy as jnp
from jax.experimental import pallas as pl

_BLOCK = 4096


def _router_kernel(x_ref, w1_ref, b1_ref, w2_ref, b2_ref, idx_ref, prob_ref):
    x = x_ref[...]
    h = jnp.maximum(
        jnp.dot(x, w1_ref[...], preferred_element_type=jnp.float32) + b1_ref[...],
        0.0,
    )
    s = jnp.dot(h, w2_ref[...], preferred_element_type=jnp.float32) + b2_ref[...]

    e = float(s.shape[1])
    lane = jax.lax.broadcasted_iota(jnp.int32, s.shape, 1).astype(jnp.float32)

    m0 = jnp.max(s, axis=1, keepdims=True)
    i0 = jnp.min(jnp.where(s == m0, lane, e), axis=1, keepdims=True)
    s_masked = jnp.where(lane == i0, -jnp.inf, s)
    m1 = jnp.max(s_masked, axis=1, keepdims=True)
    i1 = jnp.min(jnp.where(s_masked == m1, lane, e), axis=1, keepdims=True)

    denom = jnp.sum(jnp.exp(s - m0), axis=1, keepdims=True)
    p0 = 1.0 / denom
    p1 = jnp.exp(m1 - m0) / denom

    idx_ref[...] = jnp.concatenate([i0, i1], axis=1).astype(jnp.int32)
    prob_ref[...] = jnp.concatenate([p0, p1], axis=1)


def kernel(x, W1, b1, W2, b2):
    n = x.shape[0]
    grid = n // _BLOCK
    return pl.pallas_call(
        _router_kernel,
        grid=(grid,),
        in_specs=[
            pl.BlockSpec((_BLOCK, x.shape[1]), lambda i: (i, 0)),
            pl.BlockSpec(W1.shape, lambda i: (0, 0)),
            pl.BlockSpec(b1.shape, lambda i: (0,)),
            pl.BlockSpec(W2.shape, lambda i: (0, 0)),
            pl.BlockSpec(b2.shape, lambda i: (0,)),
        ],
        out_specs=[
            pl.BlockSpec((_BLOCK, 2), lambda i: (i, 0)),
            pl.BlockSpec((_BLOCK, 2), lambda i: (i, 0)),
        ],
        out_shape=[
            jax.ShapeDtypeStruct((n, 2), jnp.int32),
            jax.ShapeDtypeStruct((n, 2), jnp.float32),
        ],
    )(x, W1, b1, W2, b2)
